# baseline (speedup 1.0000x reference)
import jax
import jax.numpy as jnp
from jax import lax
from jax.experimental import pallas as pl
from jax.experimental.pallas import tpu as pltpu
from jax.experimental.pallas import tpu_sc as plsc

D = 128
S = 8192
C = 26
B = 16384
L = 16
CHUNK = 128
NV = B // L


def _body(emb, lab, ptrs, banks, out, ptrspad,
          lab_v, src_v, rows_v, idx_src, idx_w, ptr_v, scr16,
          copy_sem, gat_sem, sct_sem):
    nc = 2
    wid = lax.axis_index("s") * nc + lax.axis_index("c")

    @pl.when(wid < C)
    def _work():
        c = wid
        cp = pltpu.make_async_copy(banks.at[pl.ds(c * S, S)],
                                   out.at[pl.ds(c * S, S)], copy_sem)
        cp.start()

        pltpu.sync_copy(lab, lab_v)
        pltpu.sync_copy(ptrs, ptr_v)
        cvec = jnp.full((L,), c, jnp.int32)
        pvec = plsc.load_gather(ptr_v, [cvec])
        lane = lax.iota(jnp.int32, L)

        def scan_body(v, cur):
            l16 = lab_v[pl.ds(v * L, L)]
            m = l16 == cvec
            ids = lane + v * L
            plsc.store_compressed(src_v.at[pl.ds(cur, L)], ids, mask=m)
            return cur + jnp.sum(m.astype(jnp.int32))

        cnt = lax.fori_loop(0, NV, scan_body, 0, unroll=4)

        scr16[...] = (pvec + cnt) & (S - 1)
        pltpu.sync_copy(scr16, ptrspad.at[c])

        cp.wait()

        nch = (cnt + CHUNK - 1) // CHUNK

        def chunk_body(t, carry):
            k0 = t * CHUNK
            for t2 in range(CHUNK // L):
                j = k0 + t2 * L + lane
                je = jnp.minimum(j, cnt - 1)
                sidx = plsc.load_gather(src_v, [je])
                idx_src[pl.ds(t2 * L, L)] = sidx
                idx_w[0, pl.ds(t2 * L, L)] = c * S + ((pvec + je) & (S - 1))
            g = pltpu.make_async_copy(emb.at[idx_src], rows_v, gat_sem)
            g.start()
            g.wait()
            sc = pltpu.make_async_copy(rows_v, out.at[idx_w.at[0]], sct_sem)
            sc.start()
            sc.wait()
            return carry

        lax.fori_loop(0, nch, chunk_body, 0)


_sc_call = pl.kernel(
    _body,
    out_type=[
        jax.ShapeDtypeStruct((C * S, D), jnp.float32),
        jax.ShapeDtypeStruct((C, L), jnp.int32),
    ],
    mesh=plsc.VectorSubcoreMesh(core_axis_name="c", subcore_axis_name="s"),
    compiler_params=pltpu.CompilerParams(needs_layout_passes=False),
    scratch_types=[
        pltpu.VMEM((B,), jnp.int32),
        pltpu.VMEM((B,), jnp.int32),
        pltpu.VMEM((CHUNK, D), jnp.float32),
        pltpu.VMEM((CHUNK,), jnp.int32),
        pltpu.VMEM((1, CHUNK), jnp.int32),
        pltpu.VMEM((32,), jnp.int32),
        pltpu.VMEM((L,), jnp.int32),
        pltpu.SemaphoreType.DMA,
        pltpu.SemaphoreType.DMA,
        pltpu.SemaphoreType.DMA,
    ],
)


def kernel(embeddings, labels, banks, ptrs):
    banks_flat = banks.reshape(C * S, D)
    ptrs_pad = jnp.pad(ptrs, (0, 32 - C))
    out_flat, ptrspad = _sc_call(embeddings, labels, ptrs_pad, banks_flat)
    return out_flat.reshape(C, S, D), ptrspad[:, 0]
